# baseline (speedup 1.0000x reference)
import functools

import jax
import jax.numpy as jnp
from jax import lax
from jax.experimental import pallas as pl
from jax.experimental.pallas import tpu as pltpu
from jax.experimental.pallas import tpu_sc as plsc

N = 50000
E = 800000
H = 32
NLAYERS = 4

NC = 2
NS = 16
NW = NC * NS
CB = 1024
NSTEP = 25
EPAD = NW * CB * NSTEP
NPAD = 51200
RPS = NPAD // NS
DUMMY = N

BN = 256
BE = 2048
f32 = jnp.float32

_mesh = plsc.VectorSubcoreMesh(core_axis_name="c", subcore_axis_name="s")
_sc_params = pltpu.CompilerParams(use_tc_tiling_on_sc=False)
_sc_params_scatter = pltpu.CompilerParams(
    use_tc_tiling_on_sc=False, internal_scratch_in_bytes=0)


EPW = EPAD // NW
ROWS = EPW // 128


def _make_gather(W, cb=256):
    nv = W // 16
    nblk = cb // 128
    nstep = EPW // cb
    nsup = nstep // 4

    @functools.partial(
        pl.kernel,
        out_type=jax.ShapeDtypeStruct((EPAD, W), f32),
        mesh=_mesh,
        compiler_params=_sc_params,
        scratch_types=[
            pltpu.VMEM((4, 2 * cb), jnp.int32),
            pltpu.VMEM((2, cb, W), f32),
            pltpu.VMEM((2, cb, W), f32),
        ] + [pltpu.SemaphoreType.DMA] * 8,
    )
    def gather(atbl, btbl, rci, out, idx, bufa, bufb,
               si0, si1, si2, si3, sg0, sg1, so0, so1):
        cid = lax.axis_index("c")
        sid = lax.axis_index("s")
        wid = sid * NC + cid
        sem_i = (si0, si1, si2, si3)
        sem_g = (sg0, sg1)
        sem_o = (so0, so1)

        def fire_idx(g, q):
            src = pl.ds((wid * nstep + g) * 2 * cb, 2 * cb)
            pltpu.async_copy(rci.at[src], idx.at[q], sem_i[q])

        def wait_idx(q):
            pltpu.make_async_copy(rci.at[pl.ds(0, 2 * cb)], idx.at[q],
                                  sem_i[q]).wait()

        def fire(g, p, q):
            pltpu.async_copy(atbl.at[idx.at[q, pl.ds(0, cb)]],
                             bufa.at[p], sem_g[p])
            pltpu.async_copy(btbl.at[idx.at[q, pl.ds(cb, cb)]],
                             bufb.at[p], sem_g[p])

        def wait_gathers(p):
            pltpu.make_async_copy(atbl.at[pl.ds(0, cb)],
                                  bufa.at[p], sem_g[p]).wait()
            pltpu.make_async_copy(btbl.at[pl.ds(0, cb)],
                                  bufb.at[p], sem_g[p]).wait()

        def add_compute(p):
            def add_fn(r, c2):
                for k in range(nv):
                    sl = pl.ds(k * 16, 16)
                    bufa[p, r, sl] = bufa[p, r, sl] + bufb[p, r, sl]
                return c2
            lax.fori_loop(0, cb, add_fn, 0, unroll=8)

        def fire_out(g, p):
            pltpu.async_copy(bufa.at[p],
                             out.at[pl.ds(wid * EPW + g * cb, cb)], sem_o[p])

        def wait_out(p):
            pltpu.make_async_copy(bufa.at[p], out.at[pl.ds(0, cb)],
                                  sem_o[p]).wait()

        fire_idx(0, 0)
        fire_idx(1, 1)

        def body(s, carry):
            for u in range(4):
                g = 4 * s + u
                p = u & 1
                qn = (u + 2) % 4
                if u < 2:
                    fire_idx(g + 2, qn)
                else:
                    @pl.when(s < nsup - 1)
                    def _():
                        fire_idx(g + 2, qn)
                wait_idx(u)
                if u < 2:
                    @pl.when(s >= 1)
                    def _():
                        wait_out(p)
                else:
                    wait_out(p)
                fire(g, p, u)
                if u == 0:
                    @pl.when(s >= 1)
                    def _():
                        wait_gathers(1 - p)
                        add_compute(1 - p)
                        fire_out(g - 1, 1 - p)
                else:
                    wait_gathers(1 - p)
                    add_compute(1 - p)
                    fire_out(g - 1, 1 - p)
            return carry

        lax.fori_loop(0, nsup, body, 0)
        wait_gathers(1)
        add_compute(1)
        fire_out(nstep - 1, 1)
        wait_out(0)
        wait_out(1)

    return gather


_gather48 = _make_gather(48)
_gather32 = _make_gather(32)


HALF = NPAD // NC
ACC_ROWS = 26112
ZR = ACC_ROWS // NS
OR_ = HALF // NS
SPS = EPAD // NS
NSTEP2 = SPS // CB


SCB = 512
SNB = SCB // 128
SSTEP = SPS // SCB
SRING = 4
SSUP = SSTEP // SRING


@functools.partial(
    pl.kernel,
    out_type=jax.ShapeDtypeStruct((NPAD, H), f32),
    mesh=_mesh,
    compiler_params=_sc_params_scatter,
    scratch_types=[
        pltpu.VMEM((SRING, SCB), jnp.int32),
        pltpu.VMEM((SRING, SCB, H), f32),
        pltpu.VMEM_SHARED((ACC_ROWS, H), f32),
    ] + [pltpu.SemaphoreType.DMA] * 12,
)
def _scatter(ef, rowi, zrows, pout, idx, bufe, acc,
             si0, si1, si2, si3, se0, se1, se2, se3, ss0, ss1, ss2, ss3):
    cid = lax.axis_index("c")
    sid = lax.axis_index("s")
    base0 = cid * HALF
    sem_i = (si0, si1, si2, si3)
    sem_e = (se0, se1, se2, se3)
    sem_s = (ss0, ss1, ss2, ss3)

    pltpu.sync_copy(zrows, acc.at[pl.ds(sid * ZR, ZR)])
    plsc.subcore_barrier()

    def fire_idx(g, slot):
        src = pl.ds(sid * SPS + g * SCB, SCB)
        pltpu.async_copy(rowi.at[src], idx.at[slot], sem_i[slot])

    def wait_idx(slot):
        pltpu.make_async_copy(rowi.at[pl.ds(0, SCB)], idx.at[slot],
                              sem_i[slot]).wait()

    def localize(slot):
        def loc_fn(t, c2):
            cc = t * 16
            v = idx[slot, pl.ds(cc, 16)] - base0
            bad = (v < 0) | (v >= HALF)
            idx[slot, pl.ds(cc, 16)] = jnp.where(bad, HALF, v)
            return c2
        lax.fori_loop(0, SCB // 16, loc_fn, 0, unroll=8)

    def fire_load(g, slot):
        pltpu.async_copy(ef.at[pl.ds(sid * SPS + g * SCB, SCB)],
                         bufe.at[slot], sem_e[slot])

    def wait_load(slot):
        pltpu.make_async_copy(ef.at[pl.ds(0, SCB)], bufe.at[slot],
                              sem_e[slot]).wait()

    def fire_scatter(g, slot):
        pltpu.async_copy(bufe.at[slot], acc.at[idx.at[slot]], sem_s[slot],
                         add=True)

    def wait_scatter(slot):
        pltpu.make_async_copy(bufe.at[slot], acc.at[pl.ds(0, SCB)],
                              sem_s[slot]).wait()

    fire_idx(0, 0)
    fire_load(0, 0)
    fire_idx(1, 1)
    fire_load(1, 1)

    def body(s, carry):
        for u in range(SRING):
            g = SRING * s + u
            nxt2 = (u + 2) % SRING
            if u < 2:
                @pl.when(s >= 1)
                def _():
                    wait_scatter(nxt2)
                fire_idx(g + 2, nxt2)
                fire_load(g + 2, nxt2)
            else:
                wait_scatter(nxt2)

                @pl.when(s < SSUP - 1)
                def _():
                    fire_idx(g + 2, nxt2)
                    fire_load(g + 2, nxt2)
            wait_idx(u)
            localize(u)
            wait_load(u)
            fire_scatter(g, u)
        return carry

    lax.fori_loop(0, SSUP, body, 0)
    wait_scatter(2)
    wait_scatter(3)
    plsc.subcore_barrier()
    pltpu.sync_copy(acc.at[pl.ds(sid * OR_, OR_)],
                    pout.at[pl.ds(base0 + sid * OR_, OR_)])


def _enc_call(feats, pos4, encW, enc_b, eW1a, eW1b, eb1):
    def body(f_ref, p_ref, w_ref, b_ref, wa_ref, wb_ref, b1_ref,
             h_ref, a_ref, bt_ref):
        h = f_ref[:] @ w_ref[:] + b_ref[:]
        h_ref[:] = h
        a = h @ wa_ref[:] + b1_ref[:]
        b = h @ wb_ref[:]
        p = p_ref[:]
        z = jnp.zeros((BN, 12), f32)
        a_ref[:] = jnp.concatenate([a, p, z], axis=1)
        bt_ref[:] = jnp.concatenate([b, -p, z], axis=1)

    return pl.pallas_call(
        body,
        grid=(NPAD // BN,),
        in_specs=[
            pl.BlockSpec((BN, 40), lambda i: (i, 0)),
            pl.BlockSpec((BN, 4), lambda i: (i, 0)),
            pl.BlockSpec((40, H), lambda i: (0, 0)),
            pl.BlockSpec((1, H), lambda i: (0, 0)),
            pl.BlockSpec((H, H), lambda i: (0, 0)),
            pl.BlockSpec((H, H), lambda i: (0, 0)),
            pl.BlockSpec((1, H), lambda i: (0, 0)),
        ],
        out_specs=[
            pl.BlockSpec((BN, H), lambda i: (i, 0)),
            pl.BlockSpec((BN, 48), lambda i: (i, 0)),
            pl.BlockSpec((BN, 48), lambda i: (i, 0)),
        ],
        out_shape=[
            jax.ShapeDtypeStruct((NPAD, H), f32),
            jax.ShapeDtypeStruct((NPAD, 48), f32),
            jax.ShapeDtypeStruct((NPAD, 48), f32),
        ],
    )(feats, pos4, encW, enc_b, eW1a, eW1b, eb1)


def _edge0_call(s48, w2, b2, w65):
    def body(s_ref, w2_ref, b2_ref, w65_ref, ef_ref, r_ref):
        s = s_ref[:]
        g = s[:, :H]
        d = s[:, H:H + 4]
        r = jnp.sum(d * d, axis=1, keepdims=True)
        ef1 = jax.nn.relu(g + r * w65_ref[:])
        ef_ref[:] = jax.nn.relu(ef1 @ w2_ref[:] + b2_ref[:])
        r_ref[:] = r

    return pl.pallas_call(
        body,
        grid=(EPAD // BE,),
        in_specs=[
            pl.BlockSpec((BE, 48), lambda i: (i, 0)),
            pl.BlockSpec((H, H), lambda i: (0, 0)),
            pl.BlockSpec((1, H), lambda i: (0, 0)),
            pl.BlockSpec((1, H), lambda i: (0, 0)),
        ],
        out_specs=[
            pl.BlockSpec((BE, H), lambda i: (i, 0)),
            pl.BlockSpec((BE, 1), lambda i: (i, 0)),
        ],
        out_shape=[
            jax.ShapeDtypeStruct((EPAD, H), f32),
            jax.ShapeDtypeStruct((EPAD, 1), f32),
        ],
    )(s48, w2, b2, w65)


def _edge_call(g, radial, w2, b2, w65):
    def body(g_ref, r_ref, w2_ref, b2_ref, w65_ref, ef_ref):
        r = r_ref[:]
        ef1 = jax.nn.relu(g_ref[:] + r * w65_ref[:])
        ef_ref[:] = jax.nn.relu(ef1 @ w2_ref[:] + b2_ref[:])

    return pl.pallas_call(
        body,
        grid=(EPAD // BE,),
        in_specs=[
            pl.BlockSpec((BE, H), lambda i: (i, 0)),
            pl.BlockSpec((BE, 1), lambda i: (i, 0)),
            pl.BlockSpec((H, H), lambda i: (0, 0)),
            pl.BlockSpec((1, H), lambda i: (0, 0)),
            pl.BlockSpec((1, H), lambda i: (0, 0)),
        ],
        out_specs=pl.BlockSpec((BE, H), lambda i: (i, 0)),
        out_shape=jax.ShapeDtypeStruct((EPAD, H), f32),
    )(g, radial, w2, b2, w65)


def _node_call(h, p0, w1, b1, w2, b2, lg, lb, wa, wb, eb, last):
    def body(h_ref, p0_ref, w1_ref, b1_ref, w2_ref, b2_ref,
             g_ref, be_ref, wa_ref, wb_ref, ebi_ref, *outs):
        h_in = h_ref[:]
        agg = p0_ref[:]
        nin = jnp.concatenate([h_in, agg], axis=1)
        o = jax.nn.relu(nin @ w1_ref[:] + b1_ref[:])
        o = o @ w2_ref[:] + b2_ref[:]
        m = jnp.mean(o, axis=1, keepdims=True)
        v = jnp.mean((o - m) ** 2, axis=1, keepdims=True)
        ln = (o - m) / jnp.sqrt(v + 1e-5) * g_ref[:] + be_ref[:]
        hn = jax.nn.relu(ln) + h_in
        outs[0][:] = hn
        if not last:
            outs[1][:] = hn @ wa_ref[:] + ebi_ref[:]
            outs[2][:] = hn @ wb_ref[:]

    n_out = 1 if last else 3
    out_specs = [pl.BlockSpec((BN, H), lambda i: (i, 0)) for _ in range(n_out)]
    out_shape = [jax.ShapeDtypeStruct((NPAD, H), f32) for _ in range(n_out)]
    res = pl.pallas_call(
        body,
        grid=(NPAD // BN,),
        in_specs=[
            pl.BlockSpec((BN, H), lambda i: (i, 0)),
            pl.BlockSpec((BN, H), lambda i: (i, 0)),
            pl.BlockSpec((2 * H, H), lambda i: (0, 0)),
            pl.BlockSpec((1, H), lambda i: (0, 0)),
            pl.BlockSpec((H, H), lambda i: (0, 0)),
            pl.BlockSpec((1, H), lambda i: (0, 0)),
            pl.BlockSpec((1, H), lambda i: (0, 0)),
            pl.BlockSpec((1, H), lambda i: (0, 0)),
            pl.BlockSpec((H, H), lambda i: (0, 0)),
            pl.BlockSpec((H, H), lambda i: (0, 0)),
            pl.BlockSpec((1, H), lambda i: (0, 0)),
        ],
        out_specs=out_specs,
        out_shape=out_shape,
    )(h, p0, w1, b1, w2, b2, lg, lb, wa, wb, eb)
    return res


def kernel(x, pos, edge_attr, edge_index, batch, enc_W, enc_b, eW1, eb1,
           eW2, eb2, nW1, nb1, nW2, nb2, ln_g, ln_b):
    row = edge_index[0]
    col = edge_index[1]
    pad = jnp.full((EPAD - E,), DUMMY, jnp.int32)
    rowp = jnp.concatenate([row, pad])
    colp = jnp.concatenate([col, pad])
    rci = jnp.stack([rowp.reshape(-1, 256), colp.reshape(-1, 256)],
                    axis=1).reshape(-1)

    feats = jnp.zeros((NPAD, 40), f32)
    feats = feats.at[:N, :32].set(x).at[:N, 32:35].set(pos)
    encWp = jnp.zeros((40, H), f32).at[:35].set(enc_W)
    pos4 = jnp.zeros((NPAD, 4), f32).at[:N, :3].set(pos)
    zrows = jnp.zeros((ZR, H), f32)

    def r2(v):
        return v.reshape(1, H)

    h, atbl, btbl = _enc_call(feats, pos4, encWp, r2(enc_b),
                              eW1[0, :H], eW1[0, H:2 * H], r2(eb1[0]))

    radial = None
    for i in range(NLAYERS):
        if i == 0:
            s48 = _gather48(atbl, btbl, rci)
            ef, radial = _edge0_call(s48, eW2[0], r2(eb2[0]), r2(eW1[0, 64]))
        else:
            g = _gather32(atbl, btbl, rci)
            ef = _edge_call(g, radial, eW2[i], r2(eb2[i]), r2(eW1[i, 64]))
        agg = _scatter(ef, rowp, zrows)
        last = i == NLAYERS - 1
        if last:
            wa = wb = eW1[0, :H]
            eb = r2(eb1[0])
        else:
            wa = eW1[i + 1, :H]
            wb = eW1[i + 1, H:2 * H]
            eb = r2(eb1[i + 1])
        res = _node_call(h, agg, nW1[i], r2(nb1[i]), nW2[i], r2(nb2[i]),
                         r2(ln_g[i]), r2(ln_b[i]), wa, wb, eb, last)
        if last:
            h = res[0]
        else:
            h, atbl, btbl = res
    return h[:N]


# speedup vs baseline: 1.6242x; 1.6242x over previous
import functools

import jax
import jax.numpy as jnp
from jax import lax
from jax.experimental import pallas as pl
from jax.experimental.pallas import tpu as pltpu
from jax.experimental.pallas import tpu_sc as plsc

N = 50000
E = 800000
H = 32
NLAYERS = 4

NC = 2
NS = 16
NW = NC * NS
CB = 1024
NSTEP = 25
EPAD = NW * CB * NSTEP
NPAD = 51200
RPS = NPAD // NS
DUMMY = N

BN = 1024
BE = 8192
f32 = jnp.float32

_mesh = plsc.VectorSubcoreMesh(core_axis_name="c", subcore_axis_name="s")
_sc_params = pltpu.CompilerParams(use_tc_tiling_on_sc=False)
_sc_params_scatter = pltpu.CompilerParams(use_tc_tiling_on_sc=False)


EPW = EPAD // NW
ROWS = EPW // 128


def _make_gather(W, cb=256):
    nv = W // 16
    nblk = cb // 128
    nstep = EPW // cb
    nsup = nstep // 4

    @functools.partial(
        pl.kernel,
        out_type=jax.ShapeDtypeStruct((EPAD, W), f32),
        mesh=_mesh,
        compiler_params=_sc_params,
        scratch_types=[
            pltpu.VMEM((4, 2 * cb), jnp.int32),
            pltpu.VMEM((2, cb, W), f32),
            pltpu.VMEM((2, cb, W), f32),
        ] + [pltpu.SemaphoreType.DMA] * 8,
    )
    def gather(atbl, btbl, rci, out, idx, bufa, bufb,
               si0, si1, si2, si3, sg0, sg1, so0, so1):
        cid = lax.axis_index("c")
        sid = lax.axis_index("s")
        wid = sid * NC + cid
        sem_i = (si0, si1, si2, si3)
        sem_g = (sg0, sg1)
        sem_o = (so0, so1)

        def fire_idx(g, q):
            src = pl.ds((wid * nstep + g) * 2 * cb, 2 * cb)
            pltpu.async_copy(rci.at[src], idx.at[q], sem_i[q])

        def wait_idx(q):
            pltpu.make_async_copy(rci.at[pl.ds(0, 2 * cb)], idx.at[q],
                                  sem_i[q]).wait()

        def fire(g, p, q):
            pltpu.async_copy(atbl.at[idx.at[q, pl.ds(0, cb)]],
                             bufa.at[p], sem_g[p])
            pltpu.async_copy(btbl.at[idx.at[q, pl.ds(cb, cb)]],
                             bufb.at[p], sem_g[p])

        def wait_gathers(p):
            pltpu.make_async_copy(atbl.at[pl.ds(0, cb)],
                                  bufa.at[p], sem_g[p]).wait()
            pltpu.make_async_copy(btbl.at[pl.ds(0, cb)],
                                  bufb.at[p], sem_g[p]).wait()

        def add_compute(p):
            def add_fn(r, c2):
                for k in range(nv):
                    sl = pl.ds(k * 16, 16)
                    bufa[p, r, sl] = bufa[p, r, sl] + bufb[p, r, sl]
                return c2
            lax.fori_loop(0, cb, add_fn, 0, unroll=8)

        def fire_out(g, p):
            pltpu.async_copy(bufa.at[p],
                             out.at[pl.ds(wid * EPW + g * cb, cb)], sem_o[p])

        def wait_out(p):
            pltpu.make_async_copy(bufa.at[p], out.at[pl.ds(0, cb)],
                                  sem_o[p]).wait()

        fire_idx(0, 0)
        fire_idx(1, 1)

        def body(s, carry):
            for u in range(4):
                g = 4 * s + u
                p = u & 1
                qn = (u + 2) % 4
                if u < 2:
                    fire_idx(g + 2, qn)
                else:
                    @pl.when(s < nsup - 1)
                    def _():
                        fire_idx(g + 2, qn)
                wait_idx(u)
                if u < 2:
                    @pl.when(s >= 1)
                    def _():
                        wait_out(p)
                else:
                    wait_out(p)
                fire(g, p, u)
                if u == 0:
                    @pl.when(s >= 1)
                    def _():
                        wait_gathers(1 - p)
                        add_compute(1 - p)
                        fire_out(g - 1, 1 - p)
                else:
                    wait_gathers(1 - p)
                    add_compute(1 - p)
                    fire_out(g - 1, 1 - p)
            return carry

        lax.fori_loop(0, nsup, body, 0)
        wait_gathers(1)
        add_compute(1)
        fire_out(nstep - 1, 1)
        wait_out(0)
        wait_out(1)

    return gather


_gather48 = _make_gather(48)
_gather32 = _make_gather(32)


HALF = NPAD // NC
ACC_ROWS = 26112
ZR = ACC_ROWS // NS
OR_ = HALF // NS
SPS = EPAD // NS
NSTEP2 = SPS // CB


SCB = 1024
SNB = SCB // 128
SSTEP = SPS // SCB
SRING = 2
SSUP = SSTEP // SRING


@functools.partial(
    pl.kernel,
    out_type=jax.ShapeDtypeStruct((NPAD, H), f32),
    mesh=_mesh,
    compiler_params=_sc_params_scatter,
    scratch_types=[
        pltpu.VMEM((SRING, SNB, 128), jnp.int32),
        pltpu.VMEM((SRING, SCB, H), f32),
        pltpu.VMEM_SHARED((ACC_ROWS, H), f32),
    ] + [pltpu.SemaphoreType.DMA] * 12,
)
def _scatter(ef, rowi, zrows, pout, idx, bufe, acc,
             si0, si1, si2, si3, se0, se1, se2, se3, ss0, ss1, ss2, ss3):
    cid = lax.axis_index("c")
    sid = lax.axis_index("s")
    base0 = cid * HALF
    sem_i = (si0, si1, si2, si3)
    sem_e = (se0, se1, se2, se3)
    sem_s = (ss0, ss1, ss2, ss3)

    def m8(x):
        return pl.multiple_of(x, 8)

    pltpu.sync_copy(zrows, acc.at[pl.ds(m8(sid * ZR), ZR)])
    plsc.subcore_barrier()

    def fire_idx(g, slot):
        src = pl.ds(m8(sid * (SPS // 128) + g * SNB), SNB)
        pltpu.async_copy(rowi.at[src], idx.at[slot], sem_i[slot])

    def wait_idx(slot):
        pltpu.make_async_copy(rowi.at[pl.ds(0, SNB)], idx.at[slot],
                              sem_i[slot]).wait()

    def localize(slot):
        def loc_fn(t, c2):
            j = t // 8
            cc = (t % 8) * 16
            v = idx[slot, j, pl.ds(cc, 16)] - base0
            bad = (v < 0) | (v >= HALF)
            idx[slot, j, pl.ds(cc, 16)] = jnp.where(bad, HALF, v)
            return c2
        lax.fori_loop(0, SNB * 8, loc_fn, 0, unroll=8)

    def fire_load(g, slot):
        pltpu.async_copy(ef.at[pl.ds(m8(sid * SPS + g * SCB), SCB)],
                         bufe.at[slot], sem_e[slot])

    def wait_load(slot):
        pltpu.make_async_copy(ef.at[pl.ds(0, SCB)], bufe.at[slot],
                              sem_e[slot]).wait()

    def fire_scatter(g, slot):
        for j in range(SNB):
            pltpu.async_copy(bufe.at[slot, pl.ds(j * 128, 128)],
                             acc.at[idx.at[slot, j]], sem_s[slot], add=True)

    def wait_scatter(slot):
        for j in range(SNB):
            pltpu.make_async_copy(bufe.at[slot, pl.ds(j * 128, 128)],
                                  acc.at[pl.ds(0, 128)], sem_s[slot]).wait()

    fire_idx(0, 0)
    fire_load(0, 0)

    def body(s, carry):
        g0 = 2 * s

        @pl.when(s >= 1)
        def _():
            wait_scatter(1)
        fire_idx(g0 + 1, 1)
        fire_load(g0 + 1, 1)
        wait_idx(0)
        localize(0)
        wait_load(0)
        fire_scatter(g0, 0)
        wait_scatter(0)

        @pl.when(s < SSUP - 1)
        def _():
            fire_idx(g0 + 2, 0)
            fire_load(g0 + 2, 0)
        wait_idx(1)
        localize(1)
        wait_load(1)
        fire_scatter(g0 + 1, 1)
        return carry

    lax.fori_loop(0, SSUP, body, 0)
    wait_scatter(1)
    plsc.subcore_barrier()
    pltpu.sync_copy(acc.at[pl.ds(m8(sid * OR_), OR_)],
                    pout.at[pl.ds(m8(base0 + sid * OR_), OR_)])


def _enc_call(feats, pos4, encW, enc_b, eW1a, eW1b, eb1):
    def body(f_ref, p_ref, w_ref, b_ref, wa_ref, wb_ref, b1_ref,
             h_ref, a_ref, bt_ref):
        h = f_ref[:] @ w_ref[:] + b_ref[:]
        h_ref[:] = h
        a = h @ wa_ref[:] + b1_ref[:]
        b = h @ wb_ref[:]
        p = p_ref[:]
        z = jnp.zeros((BN, 12), f32)
        a_ref[:] = jnp.concatenate([a, p, z], axis=1)
        bt_ref[:] = jnp.concatenate([b, -p, z], axis=1)

    return pl.pallas_call(
        body,
        grid=(NPAD // BN,),
        in_specs=[
            pl.BlockSpec((BN, 40), lambda i: (i, 0)),
            pl.BlockSpec((BN, 4), lambda i: (i, 0)),
            pl.BlockSpec((40, H), lambda i: (0, 0)),
            pl.BlockSpec((1, H), lambda i: (0, 0)),
            pl.BlockSpec((H, H), lambda i: (0, 0)),
            pl.BlockSpec((H, H), lambda i: (0, 0)),
            pl.BlockSpec((1, H), lambda i: (0, 0)),
        ],
        out_specs=[
            pl.BlockSpec((BN, H), lambda i: (i, 0)),
            pl.BlockSpec((BN, 48), lambda i: (i, 0)),
            pl.BlockSpec((BN, 48), lambda i: (i, 0)),
        ],
        out_shape=[
            jax.ShapeDtypeStruct((NPAD, H), f32),
            jax.ShapeDtypeStruct((NPAD, 48), f32),
            jax.ShapeDtypeStruct((NPAD, 48), f32),
        ],
    )(feats, pos4, encW, enc_b, eW1a, eW1b, eb1)


def _edge0_call(s48, w2, b2, w65):
    def body(s_ref, w2_ref, b2_ref, w65_ref, ef_ref, r_ref):
        s = s_ref[:]
        g = s[:, :H]
        d = s[:, H:H + 4]
        r = jnp.sum(d * d, axis=1, keepdims=True)
        ef1 = jax.nn.relu(g + r * w65_ref[:])
        ef_ref[:] = jax.nn.relu(ef1 @ w2_ref[:] + b2_ref[:])
        r_ref[:] = r

    return pl.pallas_call(
        body,
        grid=(EPAD // BE,),
        in_specs=[
            pl.BlockSpec((BE, 48), lambda i: (i, 0)),
            pl.BlockSpec((H, H), lambda i: (0, 0)),
            pl.BlockSpec((1, H), lambda i: (0, 0)),
            pl.BlockSpec((1, H), lambda i: (0, 0)),
        ],
        out_specs=[
            pl.BlockSpec((BE, H), lambda i: (i, 0)),
            pl.BlockSpec((BE, 1), lambda i: (i, 0)),
        ],
        out_shape=[
            jax.ShapeDtypeStruct((EPAD, H), f32),
            jax.ShapeDtypeStruct((EPAD, 1), f32),
        ],
    )(s48, w2, b2, w65)


def _edge_call(gv, rad4, w2bd, b2w, sel):
    BV = BE // 4

    def body(g_ref, r_ref, w2_ref, b2_ref, sel_ref, ef_ref):
        ef1 = jax.nn.relu(g_ref[:] + r_ref[:] @ sel_ref[:])
        ef_ref[:] = jax.nn.relu(ef1 @ w2_ref[:] + b2_ref[:])

    return pl.pallas_call(
        body,
        grid=(EPAD // BE,),
        in_specs=[
            pl.BlockSpec((BV, 128), lambda i: (i, 0)),
            pl.BlockSpec((BV, 4), lambda i: (i, 0)),
            pl.BlockSpec((128, 128), lambda i: (0, 0)),
            pl.BlockSpec((1, 128), lambda i: (0, 0)),
            pl.BlockSpec((4, 128), lambda i: (0, 0)),
        ],
        out_specs=pl.BlockSpec((BV, 128), lambda i: (i, 0)),
        out_shape=jax.ShapeDtypeStruct((EPAD // 4, 128), f32),
    )(gv, rad4, w2bd, b2w, sel)


def _node_call(h, p0, w1, b1, w2, b2, lg, lb, wa, wb, eb, last):
    def body(h_ref, p0_ref, w1_ref, b1_ref, w2_ref, b2_ref,
             g_ref, be_ref, wa_ref, wb_ref, ebi_ref, *outs):
        h_in = h_ref[:]
        agg = p0_ref[:]
        nin = jnp.concatenate([h_in, agg], axis=1)
        o = jax.nn.relu(nin @ w1_ref[:] + b1_ref[:])
        o = o @ w2_ref[:] + b2_ref[:]
        m = jnp.mean(o, axis=1, keepdims=True)
        v = jnp.mean((o - m) ** 2, axis=1, keepdims=True)
        ln = (o - m) / jnp.sqrt(v + 1e-5) * g_ref[:] + be_ref[:]
        hn = jax.nn.relu(ln) + h_in
        outs[0][:] = hn
        if not last:
            outs[1][:] = hn @ wa_ref[:] + ebi_ref[:]
            outs[2][:] = hn @ wb_ref[:]

    n_out = 1 if last else 3
    out_specs = [pl.BlockSpec((BN, H), lambda i: (i, 0)) for _ in range(n_out)]
    out_shape = [jax.ShapeDtypeStruct((NPAD, H), f32) for _ in range(n_out)]
    res = pl.pallas_call(
        body,
        grid=(NPAD // BN,),
        in_specs=[
            pl.BlockSpec((BN, H), lambda i: (i, 0)),
            pl.BlockSpec((BN, H), lambda i: (i, 0)),
            pl.BlockSpec((2 * H, H), lambda i: (0, 0)),
            pl.BlockSpec((1, H), lambda i: (0, 0)),
            pl.BlockSpec((H, H), lambda i: (0, 0)),
            pl.BlockSpec((1, H), lambda i: (0, 0)),
            pl.BlockSpec((1, H), lambda i: (0, 0)),
            pl.BlockSpec((1, H), lambda i: (0, 0)),
            pl.BlockSpec((H, H), lambda i: (0, 0)),
            pl.BlockSpec((H, H), lambda i: (0, 0)),
            pl.BlockSpec((1, H), lambda i: (0, 0)),
        ],
        out_specs=out_specs,
        out_shape=out_shape,
    )(h, p0, w1, b1, w2, b2, lg, lb, wa, wb, eb)
    return res


def kernel(x, pos, edge_attr, edge_index, batch, enc_W, enc_b, eW1, eb1,
           eW2, eb2, nW1, nb1, nW2, nb2, ln_g, ln_b):
    row = edge_index[0]
    col = edge_index[1]
    pad = jnp.full((EPAD - E,), DUMMY, jnp.int32)
    rowp = jnp.concatenate([row, pad])
    colp = jnp.concatenate([col, pad])
    row2d = rowp.reshape(EPAD // 128, 128)
    rci = jnp.stack([rowp.reshape(-1, 256), colp.reshape(-1, 256)],
                    axis=1).reshape(-1)

    feats = jnp.zeros((NPAD, 40), f32)
    feats = feats.at[:N, :32].set(x).at[:N, 32:35].set(pos)
    encWp = jnp.zeros((40, H), f32).at[:35].set(enc_W)
    pos4 = jnp.zeros((NPAD, 4), f32).at[:N, :3].set(pos)
    zrows = jnp.zeros((ZR, H), f32)

    def r2(v):
        return v.reshape(1, H)

    h, atbl, btbl = _enc_call(feats, pos4, encWp, r2(enc_b),
                              eW1[0, :H], eW1[0, H:2 * H], r2(eb1[0]))

    radial = None
    for i in range(NLAYERS):
        if i == 0:
            s48 = _gather48(atbl, btbl, rci)
            ef, radial = _edge0_call(s48, eW2[0], r2(eb2[0]), r2(eW1[0, 64]))
        else:
            g = _gather32(atbl, btbl, rci)
            gv = g.reshape(EPAD // 4, 128)
            rad4 = radial.reshape(EPAD // 4, 4)
            w2bd = jnp.kron(jnp.eye(4, dtype=f32), eW2[i])
            sel = jnp.kron(jnp.eye(4, dtype=f32), eW1[i, 64][None, :])
            b2w = jnp.tile(eb2[i], 4).reshape(1, 128)
            efv = _edge_call(gv, rad4, w2bd, b2w, sel)
            ef = efv.reshape(EPAD, H)
        agg = _scatter(ef, row2d, zrows)
        last = i == NLAYERS - 1
        if last:
            wa = wb = eW1[0, :H]
            eb = r2(eb1[0])
        else:
            wa = eW1[i + 1, :H]
            wb = eW1[i + 1, H:2 * H]
            eb = r2(eb1[i + 1])
        res = _node_call(h, agg, nW1[i], r2(nb1[i]), nW2[i], r2(nb2[i]),
                         r2(ln_g[i]), r2(ln_b[i]), wa, wb, eb, last)
        if last:
            h = res[0]
        else:
            h, atbl, btbl = res
    return h[:N]


# speedup vs baseline: 1.8189x; 1.1199x over previous
import functools

import jax
import jax.numpy as jnp
from jax import lax
from jax.experimental import pallas as pl
from jax.experimental.pallas import tpu as pltpu
from jax.experimental.pallas import tpu_sc as plsc

N = 50000
E = 800000
H = 32
NLAYERS = 4

NC = 2
NS = 16
NW = NC * NS
CB = 1024
NSTEP = 25
EPAD = NW * CB * NSTEP
NPAD = 51200
RPS = NPAD // NS
DUMMY = N

BN = 1024
BE = 8192
f32 = jnp.float32

_mesh = plsc.VectorSubcoreMesh(core_axis_name="c", subcore_axis_name="s")
_sc_params = pltpu.CompilerParams(use_tc_tiling_on_sc=False)
_sc_params_scatter = pltpu.CompilerParams(use_tc_tiling_on_sc=False)


EPW = EPAD // NW
ROWS = EPW // 128


def _make_gather(W, cb=256):
    nv = W // 16
    nblk = cb // 128
    nstep = EPW // cb
    nsup = nstep // 4

    @functools.partial(
        pl.kernel,
        out_type=jax.ShapeDtypeStruct((EPAD, W), f32),
        mesh=_mesh,
        compiler_params=_sc_params,
        scratch_types=[
            pltpu.VMEM((4, 2 * cb), jnp.int32),
            pltpu.VMEM((2, cb, W), f32),
            pltpu.VMEM((2, cb, W), f32),
        ] + [pltpu.SemaphoreType.DMA] * 8,
    )
    def gather(atbl, btbl, rci, out, idx, bufa, bufb,
               si0, si1, si2, si3, sg0, sg1, so0, so1):
        cid = lax.axis_index("c")
        sid = lax.axis_index("s")
        wid = sid * NC + cid
        sem_i = (si0, si1, si2, si3)
        sem_g = (sg0, sg1)
        sem_o = (so0, so1)

        def fire_idx(g, q):
            src = pl.ds((wid * nstep + g) * 2 * cb, 2 * cb)
            pltpu.async_copy(rci.at[src], idx.at[q], sem_i[q])

        def wait_idx(q):
            pltpu.make_async_copy(rci.at[pl.ds(0, 2 * cb)], idx.at[q],
                                  sem_i[q]).wait()

        def fire(g, p, q):
            pltpu.async_copy(atbl.at[idx.at[q, pl.ds(0, cb)]],
                             bufa.at[p], sem_g[p])
            pltpu.async_copy(btbl.at[idx.at[q, pl.ds(cb, cb)]],
                             bufb.at[p], sem_g[p])

        def wait_gathers(p):
            pltpu.make_async_copy(atbl.at[pl.ds(0, cb)],
                                  bufa.at[p], sem_g[p]).wait()
            pltpu.make_async_copy(btbl.at[pl.ds(0, cb)],
                                  bufb.at[p], sem_g[p]).wait()

        def add_compute(p):
            def add_fn(r, c2):
                for k in range(nv):
                    sl = pl.ds(k * 16, 16)
                    bufa[p, r, sl] = bufa[p, r, sl] + bufb[p, r, sl]
                return c2
            lax.fori_loop(0, cb, add_fn, 0, unroll=8)

        def fire_out(g, p):
            pltpu.async_copy(bufa.at[p],
                             out.at[pl.ds(wid * EPW + g * cb, cb)], sem_o[p])

        def wait_out(p):
            pltpu.make_async_copy(bufa.at[p], out.at[pl.ds(0, cb)],
                                  sem_o[p]).wait()

        fire_idx(0, 0)
        fire_idx(1, 1)

        def body(s, carry):
            for u in range(4):
                g = 4 * s + u
                p = u & 1
                qn = (u + 2) % 4
                if u < 2:
                    fire_idx(g + 2, qn)
                else:
                    @pl.when(s < nsup - 1)
                    def _():
                        fire_idx(g + 2, qn)
                wait_idx(u)
                if u < 2:
                    @pl.when(s >= 1)
                    def _():
                        wait_out(p)
                else:
                    wait_out(p)
                fire(g, p, u)
                if u == 0:
                    @pl.when(s >= 1)
                    def _():
                        wait_gathers(1 - p)
                        add_compute(1 - p)
                        fire_out(g - 1, 1 - p)
                else:
                    wait_gathers(1 - p)
                    add_compute(1 - p)
                    fire_out(g - 1, 1 - p)
            return carry

        lax.fori_loop(0, nsup, body, 0)
        wait_gathers(1)
        add_compute(1)
        fire_out(nstep - 1, 1)
        wait_out(0)
        wait_out(1)

    return gather


_gather48 = _make_gather(48)
_gather32 = _make_gather(32)


HALF = NPAD // NC
ACC_ROWS = 26112
ZR = ACC_ROWS // NS
OR_ = HALF // NS
SPS = EPAD // NS
NSTEP2 = SPS // CB


SCB = 512
SNB = SCB // 128
SSTEP = EPW // SCB
SSUP = SSTEP // 2
ZR2 = NPAD // NS
bf16 = jnp.bfloat16


@functools.partial(
    pl.kernel,
    out_type=jax.ShapeDtypeStruct((NC * NPAD, H), bf16),
    mesh=_mesh,
    compiler_params=_sc_params_scatter,
    scratch_types=[
        pltpu.VMEM((2, SNB, 128), jnp.int32),
        pltpu.VMEM((2, SCB, H), bf16),
        pltpu.VMEM_SHARED((NPAD, H), bf16),
    ] + [pltpu.SemaphoreType.DMA] * 6,
)
def _scatter(ef, rowi, zrows, pout, idx, bufe, acc,
             si0, si1, se0, se1, ss0, ss1):
    cid = lax.axis_index("c")
    sid = lax.axis_index("s")
    wid = sid * NC + cid
    sem_i = (si0, si1)
    sem_e = (se0, se1)
    sem_s = (ss0, ss1)

    pltpu.sync_copy(zrows, acc.at[pl.ds(sid * ZR2, ZR2)])
    plsc.subcore_barrier()

    def fire_idx(g, slot):
        src = pl.ds(wid * (EPW // 128) + g * SNB, SNB)
        pltpu.async_copy(rowi.at[src], idx.at[slot], sem_i[slot])

    def wait_idx(slot):
        pltpu.make_async_copy(rowi.at[pl.ds(0, SNB)], idx.at[slot],
                              sem_i[slot]).wait()

    def fire_load(g, slot):
        pltpu.async_copy(ef.at[pl.ds(wid * EPW + g * SCB, SCB)],
                         bufe.at[slot], sem_e[slot])

    def wait_load(slot):
        pltpu.make_async_copy(ef.at[pl.ds(0, SCB)], bufe.at[slot],
                              sem_e[slot]).wait()

    def fire_scatter(g, slot):
        for j in range(SNB):
            pltpu.async_copy(bufe.at[slot, pl.ds(j * 128, 128)],
                             acc.at[idx.at[slot, j]], sem_s[slot], add=True)

    def wait_scatter(slot):
        for j in range(SNB):
            pltpu.make_async_copy(bufe.at[slot, pl.ds(j * 128, 128)],
                                  acc.at[pl.ds(0, 128)], sem_s[slot]).wait()

    fire_idx(0, 0)
    fire_load(0, 0)

    def body(s, carry):
        g0 = 2 * s

        @pl.when(s >= 1)
        def _():
            wait_scatter(1)
        fire_idx(g0 + 1, 1)
        fire_load(g0 + 1, 1)
        wait_idx(0)
        wait_load(0)
        fire_scatter(g0, 0)
        wait_scatter(0)

        @pl.when(s < SSUP - 1)
        def _():
            fire_idx(g0 + 2, 0)
            fire_load(g0 + 2, 0)
        wait_idx(1)
        wait_load(1)
        fire_scatter(g0 + 1, 1)
        return carry

    lax.fori_loop(0, SSUP, body, 0)
    wait_scatter(1)
    plsc.subcore_barrier()
    pltpu.sync_copy(acc.at[pl.ds(sid * ZR2, ZR2)],
                    pout.at[pl.ds(cid * NPAD + sid * ZR2, ZR2)])


def _enc_call(feats, pos4, encW, enc_b, eW1a, eW1b, eb1):
    def body(f_ref, p_ref, w_ref, b_ref, wa_ref, wb_ref, b1_ref,
             h_ref, a_ref, bt_ref):
        h = f_ref[:] @ w_ref[:] + b_ref[:]
        h_ref[:] = h
        a = h @ wa_ref[:] + b1_ref[:]
        b = h @ wb_ref[:]
        p = p_ref[:]
        z = jnp.zeros((BN, 12), f32)
        a_ref[:] = jnp.concatenate([a, p, z], axis=1)
        bt_ref[:] = jnp.concatenate([b, -p, z], axis=1)

    return pl.pallas_call(
        body,
        grid=(NPAD // BN,),
        in_specs=[
            pl.BlockSpec((BN, 40), lambda i: (i, 0)),
            pl.BlockSpec((BN, 4), lambda i: (i, 0)),
            pl.BlockSpec((40, H), lambda i: (0, 0)),
            pl.BlockSpec((1, H), lambda i: (0, 0)),
            pl.BlockSpec((H, H), lambda i: (0, 0)),
            pl.BlockSpec((H, H), lambda i: (0, 0)),
            pl.BlockSpec((1, H), lambda i: (0, 0)),
        ],
        out_specs=[
            pl.BlockSpec((BN, H), lambda i: (i, 0)),
            pl.BlockSpec((BN, 48), lambda i: (i, 0)),
            pl.BlockSpec((BN, 48), lambda i: (i, 0)),
        ],
        out_shape=[
            jax.ShapeDtypeStruct((NPAD, H), f32),
            jax.ShapeDtypeStruct((NPAD, 48), f32),
            jax.ShapeDtypeStruct((NPAD, 48), f32),
        ],
    )(feats, pos4, encW, enc_b, eW1a, eW1b, eb1)


def _edge0_call(s48, w2, b2, w65):
    def body(s_ref, w2_ref, b2_ref, w65_ref, ef_ref, r_ref):
        s = s_ref[:]
        g = s[:, :H]
        d = s[:, H:H + 4]
        r = jnp.sum(d * d, axis=1, keepdims=True)
        ef1 = jax.nn.relu(g + r * w65_ref[:])
        ef_ref[:] = jax.nn.relu(ef1 @ w2_ref[:] + b2_ref[:]).astype(bf16)
        r_ref[:] = r

    return pl.pallas_call(
        body,
        grid=(EPAD // BE,),
        in_specs=[
            pl.BlockSpec((BE, 48), lambda i: (i, 0)),
            pl.BlockSpec((H, H), lambda i: (0, 0)),
            pl.BlockSpec((1, H), lambda i: (0, 0)),
            pl.BlockSpec((1, H), lambda i: (0, 0)),
        ],
        out_specs=[
            pl.BlockSpec((BE, H), lambda i: (i, 0)),
            pl.BlockSpec((BE, 1), lambda i: (i, 0)),
        ],
        out_shape=[
            jax.ShapeDtypeStruct((EPAD, H), bf16),
            jax.ShapeDtypeStruct((EPAD, 1), f32),
        ],
    )(s48, w2, b2, w65)


def _edge_call(gv, rad4, w2bd, b2w, sel):
    BV = BE // 4

    def body(g_ref, r_ref, w2_ref, b2_ref, sel_ref, ef_ref):
        ef1 = jax.nn.relu(g_ref[:] + r_ref[:] @ sel_ref[:])
        ef_ref[:] = jax.nn.relu(ef1 @ w2_ref[:] + b2_ref[:]).astype(bf16)

    return pl.pallas_call(
        body,
        grid=(EPAD // BE,),
        in_specs=[
            pl.BlockSpec((BV, 128), lambda i: (i, 0)),
            pl.BlockSpec((BV, 4), lambda i: (i, 0)),
            pl.BlockSpec((128, 128), lambda i: (0, 0)),
            pl.BlockSpec((1, 128), lambda i: (0, 0)),
            pl.BlockSpec((4, 128), lambda i: (0, 0)),
        ],
        out_specs=pl.BlockSpec((BV, 128), lambda i: (i, 0)),
        out_shape=jax.ShapeDtypeStruct((EPAD // 4, 128), bf16),
    )(gv, rad4, w2bd, b2w, sel)


def _node_call(h, p0, p1, w1, b1, w2, b2, lg, lb, wa, wb, eb, last):
    def body(h_ref, p0_ref, p1_ref, w1_ref, b1_ref, w2_ref, b2_ref,
             g_ref, be_ref, wa_ref, wb_ref, ebi_ref, *outs):
        h_in = h_ref[:]
        agg = p0_ref[:].astype(f32) + p1_ref[:].astype(f32)
        nin = jnp.concatenate([h_in, agg], axis=1)
        o = jax.nn.relu(nin @ w1_ref[:] + b1_ref[:])
        o = o @ w2_ref[:] + b2_ref[:]
        m = jnp.mean(o, axis=1, keepdims=True)
        v = jnp.mean((o - m) ** 2, axis=1, keepdims=True)
        ln = (o - m) / jnp.sqrt(v + 1e-5) * g_ref[:] + be_ref[:]
        hn = jax.nn.relu(ln) + h_in
        outs[0][:] = hn
        if not last:
            outs[1][:] = hn @ wa_ref[:] + ebi_ref[:]
            outs[2][:] = hn @ wb_ref[:]

    n_out = 1 if last else 3
    out_specs = [pl.BlockSpec((BN, H), lambda i: (i, 0)) for _ in range(n_out)]
    out_shape = [jax.ShapeDtypeStruct((NPAD, H), f32) for _ in range(n_out)]
    res = pl.pallas_call(
        body,
        grid=(NPAD // BN,),
        in_specs=[
            pl.BlockSpec((BN, H), lambda i: (i, 0)),
            pl.BlockSpec((BN, H), lambda i: (i, 0)),
            pl.BlockSpec((BN, H), lambda i: (i, 0)),
            pl.BlockSpec((2 * H, H), lambda i: (0, 0)),
            pl.BlockSpec((1, H), lambda i: (0, 0)),
            pl.BlockSpec((H, H), lambda i: (0, 0)),
            pl.BlockSpec((1, H), lambda i: (0, 0)),
            pl.BlockSpec((1, H), lambda i: (0, 0)),
            pl.BlockSpec((1, H), lambda i: (0, 0)),
            pl.BlockSpec((H, H), lambda i: (0, 0)),
            pl.BlockSpec((H, H), lambda i: (0, 0)),
            pl.BlockSpec((1, H), lambda i: (0, 0)),
        ],
        out_specs=out_specs,
        out_shape=out_shape,
    )(h, p0, p1, w1, b1, w2, b2, lg, lb, wa, wb, eb)
    return res


def kernel(x, pos, edge_attr, edge_index, batch, enc_W, enc_b, eW1, eb1,
           eW2, eb2, nW1, nb1, nW2, nb2, ln_g, ln_b):
    row = edge_index[0]
    col = edge_index[1]
    pad = jnp.full((EPAD - E,), DUMMY, jnp.int32)
    rowp = jnp.concatenate([row, pad])
    colp = jnp.concatenate([col, pad])
    row2d = rowp.reshape(EPAD // 128, 128)
    rci = jnp.stack([rowp.reshape(-1, 256), colp.reshape(-1, 256)],
                    axis=1).reshape(-1)

    feats = jnp.zeros((NPAD, 40), f32)
    feats = feats.at[:N, :32].set(x).at[:N, 32:35].set(pos)
    encWp = jnp.zeros((40, H), f32).at[:35].set(enc_W)
    pos4 = jnp.zeros((NPAD, 4), f32).at[:N, :3].set(pos)
    zrows = jnp.zeros((ZR2, H), bf16)

    def r2(v):
        return v.reshape(1, H)

    h, atbl, btbl = _enc_call(feats, pos4, encWp, r2(enc_b),
                              eW1[0, :H], eW1[0, H:2 * H], r2(eb1[0]))

    radial = None
    for i in range(NLAYERS):
        if i == 0:
            s48 = _gather48(atbl, btbl, rci)
            ef, radial = _edge0_call(s48, eW2[0], r2(eb2[0]), r2(eW1[0, 64]))
        else:
            g = _gather32(atbl, btbl, rci)
            gv = g.reshape(EPAD // 4, 128)
            rad4 = radial.reshape(EPAD // 4, 4)
            w2bd = jnp.kron(jnp.eye(4, dtype=f32), eW2[i])
            sel = jnp.kron(jnp.eye(4, dtype=f32), eW1[i, 64][None, :])
            b2w = jnp.tile(eb2[i], 4).reshape(1, 128)
            efv = _edge_call(gv, rad4, w2bd, b2w, sel)
            ef = efv.reshape(EPAD, H)
        pflat = _scatter(ef, row2d, zrows)
        p0 = pflat[:NPAD]
        p1 = pflat[NPAD:]
        last = i == NLAYERS - 1
        if last:
            wa = wb = eW1[0, :H]
            eb = r2(eb1[0])
        else:
            wa = eW1[i + 1, :H]
            wb = eW1[i + 1, H:2 * H]
            eb = r2(eb1[i + 1])
        res = _node_call(h, p0, p1, nW1[i], r2(nb1[i]), nW2[i], r2(nb2[i]),
                         r2(ln_g[i]), r2(ln_b[i]), wa, wb, eb, last)
        if last:
            h = res[0]
        else:
            h, atbl, btbl = res
    return h[:N]


# speedup vs baseline: 1.9571x; 1.0760x over previous
import functools

import jax
import jax.numpy as jnp
from jax import lax
from jax.experimental import pallas as pl
from jax.experimental.pallas import tpu as pltpu
from jax.experimental.pallas import tpu_sc as plsc

N = 50000
E = 800000
H = 32
NLAYERS = 4

NC = 2
NS = 16
NW = NC * NS
CB = 1024
NSTEP = 25
EPAD = NW * CB * NSTEP
NPAD = 51200
RPS = NPAD // NS
DUMMY = N

BN = 1024
BE = 8192
f32 = jnp.float32
bf16 = jnp.bfloat16

_mesh = plsc.VectorSubcoreMesh(core_axis_name="c", subcore_axis_name="s")
_sc_params = pltpu.CompilerParams(use_tc_tiling_on_sc=False)
_sc_params_scatter = pltpu.CompilerParams(use_tc_tiling_on_sc=False)


EPW = EPAD // NW
ROWS = EPW // 128


def _make_gather(W, cb=256, dt=f32):
    sw = 16 if dt == f32 else 32
    nv = W // sw
    nblk = cb // 128
    nstep = EPW // cb
    nsup = nstep // 4

    @functools.partial(
        pl.kernel,
        out_type=jax.ShapeDtypeStruct((EPAD, W), dt),
        mesh=_mesh,
        compiler_params=_sc_params,
        scratch_types=[
            pltpu.VMEM((4, 2 * cb), jnp.int32),
            pltpu.VMEM((2, cb, W), dt),
            pltpu.VMEM((2, cb, W), dt),
        ] + [pltpu.SemaphoreType.DMA] * 8,
    )
    def gather(atbl, btbl, rci, out, idx, bufa, bufb,
               si0, si1, si2, si3, sg0, sg1, so0, so1):
        cid = lax.axis_index("c")
        sid = lax.axis_index("s")
        wid = sid * NC + cid
        sem_i = (si0, si1, si2, si3)
        sem_g = (sg0, sg1)
        sem_o = (so0, so1)

        def fire_idx(g, q):
            src = pl.ds((wid * nstep + g) * 2 * cb, 2 * cb)
            pltpu.async_copy(rci.at[src], idx.at[q], sem_i[q])

        def wait_idx(q):
            pltpu.make_async_copy(rci.at[pl.ds(0, 2 * cb)], idx.at[q],
                                  sem_i[q]).wait()

        def fire(g, p, q):
            pltpu.async_copy(atbl.at[idx.at[q, pl.ds(0, cb)]],
                             bufa.at[p], sem_g[p])
            pltpu.async_copy(btbl.at[idx.at[q, pl.ds(cb, cb)]],
                             bufb.at[p], sem_g[p])

        def wait_gathers(p):
            pltpu.make_async_copy(atbl.at[pl.ds(0, cb)],
                                  bufa.at[p], sem_g[p]).wait()
            pltpu.make_async_copy(btbl.at[pl.ds(0, cb)],
                                  bufb.at[p], sem_g[p]).wait()

        def add_compute(p):
            def add_fn(r, c2):
                for k in range(nv):
                    sl = pl.ds(k * sw, sw)
                    bufa[p, r, sl] = bufa[p, r, sl] + bufb[p, r, sl]
                return c2
            lax.fori_loop(0, cb, add_fn, 0, unroll=8)

        def fire_out(g, p):
            pltpu.async_copy(bufa.at[p],
                             out.at[pl.ds(wid * EPW + g * cb, cb)], sem_o[p])

        def wait_out(p):
            pltpu.make_async_copy(bufa.at[p], out.at[pl.ds(0, cb)],
                                  sem_o[p]).wait()

        fire_idx(0, 0)
        fire_idx(1, 1)

        def body(s, carry):
            for u in range(4):
                g = 4 * s + u
                p = u & 1
                qn = (u + 2) % 4
                if u < 2:
                    fire_idx(g + 2, qn)
                else:
                    @pl.when(s < nsup - 1)
                    def _():
                        fire_idx(g + 2, qn)
                wait_idx(u)
                if u < 2:
                    @pl.when(s >= 1)
                    def _():
                        wait_out(p)
                else:
                    wait_out(p)
                fire(g, p, u)
                if u == 0:
                    @pl.when(s >= 1)
                    def _():
                        wait_gathers(1 - p)
                        add_compute(1 - p)
                        fire_out(g - 1, 1 - p)
                else:
                    wait_gathers(1 - p)
                    add_compute(1 - p)
                    fire_out(g - 1, 1 - p)
            return carry

        lax.fori_loop(0, nsup, body, 0)
        wait_gathers(1)
        add_compute(1)
        fire_out(nstep - 1, 1)
        wait_out(0)
        wait_out(1)

    return gather


_gather48 = _make_gather(48)
_gather32 = _make_gather(32, dt=bf16)


HALF = NPAD // NC
ACC_ROWS = 26112
ZR = ACC_ROWS // NS
OR_ = HALF // NS
SPS = EPAD // NS
NSTEP2 = SPS // CB


SCB = 512
SNB = SCB // 128
SSTEP = EPW // SCB
SSUP = SSTEP // 2
ZR2 = NPAD // NS


@functools.partial(
    pl.kernel,
    out_type=jax.ShapeDtypeStruct((NC * NPAD, H), bf16),
    mesh=_mesh,
    compiler_params=_sc_params_scatter,
    scratch_types=[
        pltpu.VMEM((2, SNB, 128), jnp.int32),
        pltpu.VMEM((2, SCB, H), bf16),
        pltpu.VMEM_SHARED((NPAD, H), bf16),
    ] + [pltpu.SemaphoreType.DMA] * 6,
)
def _scatter(ef, rowi, zrows, pout, idx, bufe, acc,
             si0, si1, se0, se1, ss0, ss1):
    cid = lax.axis_index("c")
    sid = lax.axis_index("s")
    wid = sid * NC + cid
    sem_i = (si0, si1)
    sem_e = (se0, se1)
    sem_s = (ss0, ss1)

    pltpu.sync_copy(zrows, acc.at[pl.ds(sid * ZR2, ZR2)])
    plsc.subcore_barrier()

    def fire_idx(g, slot):
        src = pl.ds(wid * (EPW // 128) + g * SNB, SNB)
        pltpu.async_copy(rowi.at[src], idx.at[slot], sem_i[slot])

    def wait_idx(slot):
        pltpu.make_async_copy(rowi.at[pl.ds(0, SNB)], idx.at[slot],
                              sem_i[slot]).wait()

    def fire_load(g, slot):
        pltpu.async_copy(ef.at[pl.ds(wid * EPW + g * SCB, SCB)],
                         bufe.at[slot], sem_e[slot])

    def wait_load(slot):
        pltpu.make_async_copy(ef.at[pl.ds(0, SCB)], bufe.at[slot],
                              sem_e[slot]).wait()

    def fire_scatter(g, slot):
        for j in range(SNB):
            pltpu.async_copy(bufe.at[slot, pl.ds(j * 128, 128)],
                             acc.at[idx.at[slot, j]], sem_s[slot], add=True)

    def wait_scatter(slot):
        for j in range(SNB):
            pltpu.make_async_copy(bufe.at[slot, pl.ds(j * 128, 128)],
                                  acc.at[pl.ds(0, 128)], sem_s[slot]).wait()

    fire_idx(0, 0)
    fire_load(0, 0)

    def body(s, carry):
        g0 = 2 * s

        @pl.when(s >= 1)
        def _():
            wait_scatter(1)
        fire_idx(g0 + 1, 1)
        fire_load(g0 + 1, 1)
        wait_idx(0)
        wait_load(0)
        fire_scatter(g0, 0)
        wait_scatter(0)

        @pl.when(s < SSUP - 1)
        def _():
            fire_idx(g0 + 2, 0)
            fire_load(g0 + 2, 0)
        wait_idx(1)
        wait_load(1)
        fire_scatter(g0 + 1, 1)
        return carry

    lax.fori_loop(0, SSUP, body, 0)
    wait_scatter(1)
    plsc.subcore_barrier()
    pltpu.sync_copy(acc.at[pl.ds(sid * ZR2, ZR2)],
                    pout.at[pl.ds(cid * NPAD + sid * ZR2, ZR2)])


def _enc_call(feats, pos4, encW, enc_b, eW1a, eW1b, eb1):
    def body(f_ref, p_ref, w_ref, b_ref, wa_ref, wb_ref, b1_ref,
             h_ref, a_ref, bt_ref):
        h = f_ref[:] @ w_ref[:] + b_ref[:]
        h_ref[:] = h
        a = h @ wa_ref[:] + b1_ref[:]
        b = h @ wb_ref[:]
        p = p_ref[:]
        z = jnp.zeros((BN, 12), f32)
        a_ref[:] = jnp.concatenate([a, p, z], axis=1)
        bt_ref[:] = jnp.concatenate([b, -p, z], axis=1)

    return pl.pallas_call(
        body,
        grid=(NPAD // BN,),
        in_specs=[
            pl.BlockSpec((BN, 40), lambda i: (i, 0)),
            pl.BlockSpec((BN, 4), lambda i: (i, 0)),
            pl.BlockSpec((40, H), lambda i: (0, 0)),
            pl.BlockSpec((1, H), lambda i: (0, 0)),
            pl.BlockSpec((H, H), lambda i: (0, 0)),
            pl.BlockSpec((H, H), lambda i: (0, 0)),
            pl.BlockSpec((1, H), lambda i: (0, 0)),
        ],
        out_specs=[
            pl.BlockSpec((BN, H), lambda i: (i, 0)),
            pl.BlockSpec((BN, 48), lambda i: (i, 0)),
            pl.BlockSpec((BN, 48), lambda i: (i, 0)),
        ],
        out_shape=[
            jax.ShapeDtypeStruct((NPAD, H), f32),
            jax.ShapeDtypeStruct((NPAD, 48), f32),
            jax.ShapeDtypeStruct((NPAD, 48), f32),
        ],
    )(feats, pos4, encW, enc_b, eW1a, eW1b, eb1)


def _edge0_call(s48, w2, b2, w65):
    def body(s_ref, w2_ref, b2_ref, w65_ref, ef_ref, r_ref):
        s = s_ref[:]
        g = s[:, :H]
        d = s[:, H:H + 4]
        r = jnp.sum(d * d, axis=1, keepdims=True)
        ef1 = jax.nn.relu(g + r * w65_ref[:])
        ef_ref[:] = jax.nn.relu(ef1 @ w2_ref[:] + b2_ref[:]).astype(bf16)
        r_ref[:] = r

    return pl.pallas_call(
        body,
        grid=(EPAD // BE,),
        in_specs=[
            pl.BlockSpec((BE, 48), lambda i: (i, 0)),
            pl.BlockSpec((H, H), lambda i: (0, 0)),
            pl.BlockSpec((1, H), lambda i: (0, 0)),
            pl.BlockSpec((1, H), lambda i: (0, 0)),
        ],
        out_specs=[
            pl.BlockSpec((BE, H), lambda i: (i, 0)),
            pl.BlockSpec((BE, 1), lambda i: (i, 0)),
        ],
        out_shape=[
            jax.ShapeDtypeStruct((EPAD, H), bf16),
            jax.ShapeDtypeStruct((EPAD, 1), f32),
        ],
    )(s48, w2, b2, w65)


def _edge_call(gv, rad4, w2bd, b2w, sel):
    BV = BE // 4

    def body(g_ref, r_ref, w2_ref, b2_ref, sel_ref, ef_ref):
        ef1 = jax.nn.relu(g_ref[:].astype(f32) + r_ref[:] @ sel_ref[:])
        ef_ref[:] = jax.nn.relu(ef1 @ w2_ref[:] + b2_ref[:]).astype(bf16)

    return pl.pallas_call(
        body,
        grid=(EPAD // BE,),
        in_specs=[
            pl.BlockSpec((BV, 128), lambda i: (i, 0)),
            pl.BlockSpec((BV, 4), lambda i: (i, 0)),
            pl.BlockSpec((128, 128), lambda i: (0, 0)),
            pl.BlockSpec((1, 128), lambda i: (0, 0)),
            pl.BlockSpec((4, 128), lambda i: (0, 0)),
        ],
        out_specs=pl.BlockSpec((BV, 128), lambda i: (i, 0)),
        out_shape=jax.ShapeDtypeStruct((EPAD // 4, 128), bf16),
    )(gv, rad4, w2bd, b2w, sel)


def _node_call(h, p0, p1, w1, b1, w2, b2, lg, lb, wa, wb, eb, last):
    def body(h_ref, p0_ref, p1_ref, w1_ref, b1_ref, w2_ref, b2_ref,
             g_ref, be_ref, wa_ref, wb_ref, ebi_ref, *outs):
        h_in = h_ref[:]
        agg = p0_ref[:].astype(f32) + p1_ref[:].astype(f32)
        nin = jnp.concatenate([h_in, agg], axis=1)
        o = jax.nn.relu(nin @ w1_ref[:] + b1_ref[:])
        o = o @ w2_ref[:] + b2_ref[:]
        m = jnp.mean(o, axis=1, keepdims=True)
        v = jnp.mean((o - m) ** 2, axis=1, keepdims=True)
        ln = (o - m) / jnp.sqrt(v + 1e-5) * g_ref[:] + be_ref[:]
        hn = jax.nn.relu(ln) + h_in
        outs[0][:] = hn
        if not last:
            outs[1][:] = (hn @ wa_ref[:] + ebi_ref[:]).astype(bf16)
            outs[2][:] = (hn @ wb_ref[:]).astype(bf16)

    n_out = 1 if last else 3
    out_specs = [pl.BlockSpec((BN, H), lambda i: (i, 0)) for _ in range(n_out)]
    out_shape = [jax.ShapeDtypeStruct((NPAD, H), f32)] + \
        [jax.ShapeDtypeStruct((NPAD, H), bf16) for _ in range(n_out - 1)]
    res = pl.pallas_call(
        body,
        grid=(NPAD // BN,),
        in_specs=[
            pl.BlockSpec((BN, H), lambda i: (i, 0)),
            pl.BlockSpec((BN, H), lambda i: (i, 0)),
            pl.BlockSpec((BN, H), lambda i: (i, 0)),
            pl.BlockSpec((2 * H, H), lambda i: (0, 0)),
            pl.BlockSpec((1, H), lambda i: (0, 0)),
            pl.BlockSpec((H, H), lambda i: (0, 0)),
            pl.BlockSpec((1, H), lambda i: (0, 0)),
            pl.BlockSpec((1, H), lambda i: (0, 0)),
            pl.BlockSpec((1, H), lambda i: (0, 0)),
            pl.BlockSpec((H, H), lambda i: (0, 0)),
            pl.BlockSpec((H, H), lambda i: (0, 0)),
            pl.BlockSpec((1, H), lambda i: (0, 0)),
        ],
        out_specs=out_specs,
        out_shape=out_shape,
    )(h, p0, p1, w1, b1, w2, b2, lg, lb, wa, wb, eb)
    return res


def kernel(x, pos, edge_attr, edge_index, batch, enc_W, enc_b, eW1, eb1,
           eW2, eb2, nW1, nb1, nW2, nb2, ln_g, ln_b):
    row = edge_index[0]
    col = edge_index[1]
    pad = jnp.full((EPAD - E,), DUMMY, jnp.int32)
    rowp = jnp.concatenate([row, pad])
    colp = jnp.concatenate([col, pad])
    row2d = rowp.reshape(EPAD // 128, 128)
    rci = jnp.stack([rowp.reshape(-1, 256), colp.reshape(-1, 256)],
                    axis=1).reshape(-1)

    feats = jnp.zeros((NPAD, 40), f32)
    feats = feats.at[:N, :32].set(x).at[:N, 32:35].set(pos)
    encWp = jnp.zeros((40, H), f32).at[:35].set(enc_W)
    pos4 = jnp.zeros((NPAD, 4), f32).at[:N, :3].set(pos)
    zrows = jnp.zeros((ZR2, H), bf16)

    def r2(v):
        return v.reshape(1, H)

    h, atbl, btbl = _enc_call(feats, pos4, encWp, r2(enc_b),
                              eW1[0, :H], eW1[0, H:2 * H], r2(eb1[0]))

    radial = None
    for i in range(NLAYERS):
        if i == 0:
            s48 = _gather48(atbl, btbl, rci)
            ef, radial = _edge0_call(s48, eW2[0], r2(eb2[0]), r2(eW1[0, 64]))
            rad4 = radial.reshape(EPAD // 4, 4)
        else:
            g = _gather32(atbl, btbl, rci)
            gv = g.reshape(EPAD // 4, 128)
            w2bd = jnp.kron(jnp.eye(4, dtype=f32), eW2[i])
            sel = jnp.kron(jnp.eye(4, dtype=f32), eW1[i, 64][None, :])
            b2w = jnp.tile(eb2[i], 4).reshape(1, 128)
            efv = _edge_call(gv, rad4, w2bd, b2w, sel)
            ef = efv.reshape(EPAD, H)
        pflat = _scatter(ef, row2d, zrows)
        p0 = pflat[:NPAD]
        p1 = pflat[NPAD:]
        last = i == NLAYERS - 1
        if last:
            wa = wb = eW1[0, :H]
            eb = r2(eb1[0])
        else:
            wa = eW1[i + 1, :H]
            wb = eW1[i + 1, H:2 * H]
            eb = r2(eb1[i + 1])
        res = _node_call(h, p0, p1, nW1[i], r2(nb1[i]), nW2[i], r2(nb2[i]),
                         r2(ln_g[i]), r2(ln_b[i]), wa, wb, eb, last)
        if last:
            h = res[0]
        else:
            h, atbl, btbl = res
    return h[:N]
